# baseline (speedup 1.0000x reference)
import functools

import jax
import jax.numpy as jnp
from jax import lax
from jax.experimental import pallas as pl
from jax.experimental.pallas import tpu as pltpu
from jax.experimental.pallas import tpu_sc as plsc

VOCAB = 1_000_000
DIM = 64
PK = DIM // 2
BBLK = 128


@functools.partial(jax.jit, static_argnums=(2, 3, 4, 5))
def _sc_embed(idx3, packed, nw, nc, nb, nf):
    mesh = plsc.VectorSubcoreMesh(core_axis_name="c", subcore_axis_name="s")
    bpw = nb // nw
    blocks = bpw // BBLK
    tiles = nf * blocks

    @functools.partial(
        pl.kernel,
        mesh=mesh,
        out_type=jax.ShapeDtypeStruct((nf, 8, nb // BBLK, 8, BBLK), jnp.float32),
        scratch_types=[
            pltpu.VMEM((nf, blocks, BBLK), jnp.int32),
            pltpu.VMEM((2, BBLK, PK), jnp.int32),
            pltpu.VMEM((2, 8, 8, BBLK), jnp.float32),
            pltpu.SemaphoreType.DMA,
            pltpu.SemaphoreType.DMA,
            pltpu.SemaphoreType.DMA,
            pltpu.SemaphoreType.DMA,
        ],
        compiler_params=pltpu.CompilerParams(
            use_tc_tiling_on_sc=False, needs_layout_passes=False
        ),
    )
    def body(idx_hbm, tab_hbm, out_hbm, idx_v, rows_v, tile_v, g0, g1, s0, s1):
        wid = lax.axis_index("s") * nc + lax.axis_index("c")
        pltpu.sync_copy(idx_hbm.at[wid], idx_v)
        gsem = (g0, g1)
        ssem = (s0, s1)
        lane = lax.iota(jnp.int32, 16)
        rowsel = [lb * 16 + lane for lb in range(8)]
        colsel = [jnp.full((16,), c, jnp.int32) for c in range(PK)]

        def gather(t, p):
            return pltpu.make_async_copy(
                tab_hbm.at[idx_v.at[t // blocks].at[t % blocks]],
                rows_v.at[p],
                gsem[p],
            )

        def store(t, p):
            return pltpu.make_async_copy(
                tile_v.at[p],
                out_hbm.at[t // blocks, :, wid * blocks + t % blocks],
                ssem[p],
            )

        def transpose(p):
            for c in range(PK):
                db, ds0 = (2 * c) // 8, (2 * c) % 8
                for lb in range(8):
                    v = plsc.load_gather(rows_v.at[p], [rowsel[lb], colsel[c]])
                    lo = plsc.bitcast(lax.shift_left(v, 16), jnp.float32)
                    hi = plsc.bitcast(v & jnp.int32(-65536), jnp.float32)
                    tile_v[p, db, ds0, pl.ds(lb * 16, 16)] = lo
                    tile_v[p, db, ds0 + 1, pl.ds(lb * 16, 16)] = hi

        gather(0, 0).start()
        gather(1, 1).start()

        def outer(g, carry):
            for p in range(2):
                t = 2 * g + p
                gather(t, p).wait()

                @pl.when(t >= 2)
                def _():
                    store(t - 2, p).wait()

                transpose(p)
                store(t, p).start()

                @pl.when(t + 2 < tiles)
                def _():
                    gather(t + 2, p).start()

            return carry

        lax.fori_loop(0, tiles // 2, outer, 0)
        store(tiles - 2, 0).wait()
        store(tiles - 1, 1).wait()

    return body(idx3, packed)


def kernel(x, weights):
    nb, nf = x.shape
    info = plsc.get_sparse_core_info()
    nw = info.num_cores * info.num_subcores
    wi = lax.bitcast_convert_type(weights, jnp.int32) + jnp.int32(0x8000)
    lo = lax.slice(wi, (0, 0), (VOCAB, DIM), (1, 2))
    hi = lax.slice(wi, (0, 1), (VOCAB, DIM), (1, 2))
    packed = (hi & jnp.int32(-65536)) | lax.shift_right_logical(lo, 16)
    bpw = nb // nw
    idx3 = (
        x.astype(jnp.int32).T.reshape(nf, nw, bpw // BBLK, BBLK).transpose(1, 0, 2, 3)
    )
    t5 = _sc_embed(idx3, packed, nw, info.num_cores, nb, nf)
    return jnp.transpose(t5, (2, 4, 0, 1, 3)).reshape(nb, nf, DIM)


# speedup vs baseline: 7.8606x; 7.8606x over previous
import functools

import jax
import jax.numpy as jnp
from jax import lax
from jax.experimental import pallas as pl
from jax.experimental.pallas import tpu as pltpu
from jax.experimental.pallas import tpu_sc as plsc

VOCAB = 1_000_000
DIM = 64
BBLK = 128


@functools.partial(jax.jit, static_argnums=(2, 3, 4, 5))
def _sc_embed(idx3, table, nw, nc, nb, nf):
    mesh = plsc.VectorSubcoreMesh(core_axis_name="c", subcore_axis_name="s")
    bpw = nb // nw
    blocks = bpw // BBLK
    tiles = nf * blocks

    @functools.partial(
        pl.kernel,
        mesh=mesh,
        out_type=jax.ShapeDtypeStruct((nf, 8, nb // BBLK, 8, BBLK), jnp.float32),
        scratch_types=[
            pltpu.VMEM((nf, blocks, BBLK), jnp.int32),
            pltpu.VMEM((2, BBLK, DIM), jnp.float32),
            pltpu.VMEM((2, 8, 8, BBLK), jnp.float32),
            pltpu.SemaphoreType.DMA,
            pltpu.SemaphoreType.DMA,
            pltpu.SemaphoreType.DMA,
            pltpu.SemaphoreType.DMA,
        ],
        compiler_params=pltpu.CompilerParams(
            use_tc_tiling_on_sc=False, needs_layout_passes=False
        ),
    )
    def body(idx_hbm, tab_hbm, out_hbm, idx_v, rows_v, tile_v, g0, g1, s0, s1):
        wid = lax.axis_index("s") * nc + lax.axis_index("c")
        pltpu.sync_copy(idx_hbm.at[wid], idx_v)
        gsem = (g0, g1)
        ssem = (s0, s1)
        lane = lax.iota(jnp.int32, 16)
        rowsel = [lb * 16 + lane for lb in range(8)]
        colsel = [jnp.full((16,), d, jnp.int32) for d in range(DIM)]

        def gather(t, p):
            return pltpu.make_async_copy(
                tab_hbm.at[idx_v.at[t // blocks].at[t % blocks]],
                rows_v.at[p],
                gsem[p],
            )

        def store(t, p):
            return pltpu.make_async_copy(
                tile_v.at[p],
                out_hbm.at[t // blocks, :, wid * blocks + t % blocks],
                ssem[p],
            )

        def transpose(p):
            for d in range(DIM):
                db, ds = d // 8, d % 8
                for lb in range(8):
                    v = plsc.load_gather(rows_v.at[p], [rowsel[lb], colsel[d]])
                    tile_v[p, db, ds, pl.ds(lb * 16, 16)] = v

        gather(0, 0).start()
        gather(1, 1).start()

        def outer(g, carry):
            for p in range(2):
                t = 2 * g + p
                gather(t, p).wait()

                @pl.when(t >= 2)
                def _():
                    store(t - 2, p).wait()

                transpose(p)
                store(t, p).start()

                @pl.when(t + 2 < tiles)
                def _():
                    gather(t + 2, p).start()

            return carry

        lax.fori_loop(0, tiles // 2, outer, 0)
        store(tiles - 2, 0).wait()
        store(tiles - 1, 1).wait()

    return body(idx3, table)


def kernel(x, weights):
    nb, nf = x.shape
    info = plsc.get_sparse_core_info()
    nw = info.num_cores * info.num_subcores
    bpw = nb // nw
    idx3 = (
        x.astype(jnp.int32).T.reshape(nf, nw, bpw // BBLK, BBLK).transpose(1, 0, 2, 3)
    )
    t5 = _sc_embed(idx3, weights, nw, info.num_cores, nb, nf)
    return jnp.transpose(t5, (2, 4, 0, 1, 3)).reshape(nb, nf, DIM)


# speedup vs baseline: 12.4873x; 1.5886x over previous
import functools

import jax
import jax.numpy as jnp
from jax import lax
from jax.experimental import pallas as pl
from jax.experimental.pallas import tpu as pltpu
from jax.experimental.pallas import tpu_sc as plsc

VOCAB = 1_000_000
DIM = 64
BBLK = 128


@functools.partial(jax.jit, static_argnums=(2, 3, 4, 5))
def _sc_embed(idx3, table, nw, nc, nb, nf):
    mesh = plsc.VectorSubcoreMesh(core_axis_name="c", subcore_axis_name="s")
    bpw = nb // nw
    blocks = bpw // BBLK
    tiles = nf * blocks

    @functools.partial(
        pl.kernel,
        mesh=mesh,
        out_type=jax.ShapeDtypeStruct((nf, 8, nb // BBLK, 8, BBLK), jnp.float32),
        scratch_types=[
            pltpu.VMEM((nf, blocks, BBLK), jnp.int32),
            pltpu.VMEM((2, BBLK, DIM), jnp.float32),
            pltpu.VMEM((2, 8, 8, BBLK), jnp.float32),
            pltpu.SemaphoreType.DMA,
            pltpu.SemaphoreType.DMA,
            pltpu.SemaphoreType.DMA,
            pltpu.SemaphoreType.DMA,
        ],
        compiler_params=pltpu.CompilerParams(
            use_tc_tiling_on_sc=False, needs_layout_passes=False
        ),
    )
    def body(idx_hbm, tab_hbm, out_hbm, idx_v, rows_v, tile_v, g0, g1, s0, s1):
        wid = lax.axis_index("s") * nc + lax.axis_index("c")
        pltpu.sync_copy(idx_hbm.at[wid], idx_v)
        gsem = (g0, g1)
        ssem = (s0, s1)
        lane = lax.iota(jnp.int32, 16)
        rowsel = [lb * 16 + lane for lb in range(8)]
        rot = [(lane + r) & jnp.int32(15) for r in range(16)]

        def gather(t, p):
            return pltpu.make_async_copy(
                tab_hbm.at[idx_v.at[t // blocks].at[t % blocks]],
                rows_v.at[p],
                gsem[p],
            )

        def store(t, p):
            return pltpu.make_async_copy(
                tile_v.at[p],
                out_hbm.at[t // blocks, :, wid * blocks + t % blocks],
                ssem[p],
            )

        def transpose(p):
            def cb_body(cb, carry):
                for r in range(16):
                    d_loc = rot[r] + cb * 16
                    db = lax.shift_right_logical(d_loc, 3)
                    ds = d_loc & jnp.int32(7)
                    for lb in range(8):
                        v = plsc.load_gather(rows_v.at[p], [rowsel[lb], d_loc])
                        plsc.store_scatter(tile_v.at[p], [db, ds, rowsel[lb]], v)
                return carry

            lax.fori_loop(0, 4, cb_body, 0)

        gather(0, 0).start()
        gather(1, 1).start()

        def outer(g, carry):
            for p in range(2):
                t = 2 * g + p
                gather(t, p).wait()

                @pl.when(t >= 2)
                def _():
                    store(t - 2, p).wait()

                transpose(p)
                store(t, p).start()

                @pl.when(t + 2 < tiles)
                def _():
                    gather(t + 2, p).start()

            return carry

        lax.fori_loop(0, tiles // 2, outer, 0)
        store(tiles - 2, 0).wait()
        store(tiles - 1, 1).wait()

    return body(idx3, table)


def kernel(x, weights):
    nb, nf = x.shape
    info = plsc.get_sparse_core_info()
    nw = info.num_cores * info.num_subcores
    bpw = nb // nw
    idx3 = (
        x.astype(jnp.int32).T.reshape(nf, nw, bpw // BBLK, BBLK).transpose(1, 0, 2, 3)
    )
    t5 = _sc_embed(idx3, weights, nw, info.num_cores, nb, nf)
    return jnp.transpose(t5, (2, 4, 0, 1, 3)).reshape(nb, nf, DIM)


# speedup vs baseline: 12.6587x; 1.0137x over previous
import functools

import jax
import jax.numpy as jnp
from jax import lax
from jax.experimental import pallas as pl
from jax.experimental.pallas import tpu as pltpu
from jax.experimental.pallas import tpu_sc as plsc

VOCAB = 1_000_000
DIM = 64
BBLK = 128


@functools.partial(jax.jit, static_argnums=(2, 3, 4, 5))
def _sc_embed(idx3, table, nw, nc, nb, nf):
    mesh = plsc.VectorSubcoreMesh(core_axis_name="c", subcore_axis_name="s")
    bpw = nb // nw
    blocks = bpw // BBLK
    tiles = nf * blocks

    @functools.partial(
        pl.kernel,
        mesh=mesh,
        out_type=jax.ShapeDtypeStruct((nf, 8, nb // BBLK, 8, BBLK), jnp.float32),
        scratch_types=[
            pltpu.VMEM((nf, blocks, BBLK), jnp.int32),
            pltpu.VMEM((4, BBLK, DIM), jnp.float32),
            pltpu.VMEM((4, 8, 8, BBLK), jnp.float32),
            pltpu.SemaphoreType.DMA,
            pltpu.SemaphoreType.DMA,
            pltpu.SemaphoreType.DMA,
            pltpu.SemaphoreType.DMA,
            pltpu.SemaphoreType.DMA,
            pltpu.SemaphoreType.DMA,
            pltpu.SemaphoreType.DMA,
            pltpu.SemaphoreType.DMA,
        ],
        compiler_params=pltpu.CompilerParams(
            use_tc_tiling_on_sc=False, needs_layout_passes=False
        ),
    )
    def body(
        idx_hbm, tab_hbm, out_hbm, idx_v, rows_v, tile_v,
        g0, g1, g2, g3, s0, s1, s2, s3,
    ):
        wid = lax.axis_index("s") * nc + lax.axis_index("c")
        pltpu.sync_copy(idx_hbm.at[wid], idx_v)
        gsem = (g0, g1, g2, g3)
        ssem = (s0, s1, s2, s3)
        lane = lax.iota(jnp.int32, 16)
        rowsel = [lb * 16 + lane for lb in range(8)]
        rot = [(lane + r) & jnp.int32(15) for r in range(16)]

        def gather(t, p):
            return pltpu.make_async_copy(
                tab_hbm.at[idx_v.at[t // blocks].at[t % blocks]],
                rows_v.at[p],
                gsem[p],
            )

        def store(t, p):
            return pltpu.make_async_copy(
                tile_v.at[p],
                out_hbm.at[t // blocks, :, wid * blocks + t % blocks],
                ssem[p],
            )

        def transpose(p):
            def cb_body(cb, carry):
                for r in range(16):
                    d_loc = rot[r] + cb * 16
                    db = lax.shift_right_logical(d_loc, 3)
                    ds = d_loc & jnp.int32(7)
                    for lb in range(8):
                        v = plsc.load_gather(rows_v.at[p], [rowsel[lb], d_loc])
                        plsc.store_scatter(tile_v.at[p], [db, ds, rowsel[lb]], v)
                return carry

            lax.fori_loop(0, 4, cb_body, 0)

        for q in range(4):
            gather(q, q).start()

        def outer(g, carry):
            for q in range(4):
                t = 4 * g + q
                gather(t, q).wait()

                @pl.when(t >= 4)
                def _():
                    store(t - 4, q).wait()

                transpose(q)
                store(t, q).start()

                @pl.when(t + 4 < tiles)
                def _():
                    gather(t + 4, q).start()

            return carry

        lax.fori_loop(0, tiles // 4, outer, 0)
        for q in range(4):
            store(tiles - 4 + q, q).wait()

    return body(idx3, table)


def kernel(x, weights):
    nb, nf = x.shape
    info = plsc.get_sparse_core_info()
    nw = info.num_cores * info.num_subcores
    bpw = nb // nw
    idx3 = (
        x.astype(jnp.int32).T.reshape(nf, nw, bpw // BBLK, BBLK).transpose(1, 0, 2, 3)
    )
    t5 = _sc_embed(idx3, weights, nw, info.num_cores, nb, nf)
    return jnp.transpose(t5, (2, 4, 0, 1, 3)).reshape(nb, nf, DIM)
